# baseline (speedup 1.0000x reference)
import functools
import math

import jax
import jax.numpy as jnp
from jax import lax
from jax.experimental import pallas as pl

D_MODEL = 768
N_FEATURES = 49152
K = 32
CLIP_DATA = 10.0
TGT = math.sqrt(D_MODEL)

BT = 256
FT = 2048
NG = 6
GW = N_FEATURES // NG
NT = GW // FT
L_BINS = NG * 128


def _enc_body(x_ref, bsum_ref, w_ref, xn_ref, bmax_ref, fidx_ref):
    g = pl.program_id(1)
    tt = pl.program_id(2)
    xn = jnp.clip(x_ref[...] * TGT, -CLIP_DATA, CLIP_DATA)
    xn_ref[...] = xn
    xm = (xn - bsum_ref[...]).astype(jnp.bfloat16)
    w = w_ref[...].astype(jnp.bfloat16)
    enc = jnp.dot(xm, w, preferred_element_type=jnp.float32)
    m = enc[:, 0:128]
    a = jnp.zeros((BT, 128), jnp.int32)
    for s in range(1, 16):
        sl = enc[:, s * 128:(s + 1) * 128]
        upd = sl > m
        m = jnp.where(upd, sl, m)
        a = jnp.where(upd, s, a)
    c = lax.broadcasted_iota(jnp.int32, (BT, 128), 1)
    fidx = g * GW + (tt * 16 + a) * 128 + c

    @pl.when(tt == 0)
    def _init():
        bmax_ref[...] = m[None]
        fidx_ref[...] = fidx[None]

    @pl.when(tt != 0)
    def _acc():
        prev = bmax_ref[...]
        upd2 = m[None] > prev
        bmax_ref[...] = jnp.where(upd2, m[None], prev)
        fidx_ref[...] = jnp.where(upd2, fidx[None], fidx_ref[...])


def _encode_binmax(x, bsum, W_enc):
    B = x.shape[0]
    grid = (B // BT, NG, NT)
    return pl.pallas_call(
        _enc_body,
        grid=grid,
        in_specs=[
            pl.BlockSpec((BT, D_MODEL), lambda b, g, t: (b, 0)),
            pl.BlockSpec((1, D_MODEL), lambda b, g, t: (0, 0)),
            pl.BlockSpec((D_MODEL, FT), lambda b, g, t: (0, g * NT + t)),
        ],
        out_specs=[
            pl.BlockSpec((BT, D_MODEL), lambda b, g, t: (b, 0)),
            pl.BlockSpec((1, BT, 128), lambda b, g, t: (g, b, 0)),
            pl.BlockSpec((1, BT, 128), lambda b, g, t: (g, b, 0)),
        ],
        out_shape=[
            jax.ShapeDtypeStruct((B, D_MODEL), jnp.float32),
            jax.ShapeDtypeStruct((NG, B, 128), jnp.float32),
            jax.ShapeDtypeStruct((NG, B, 128), jnp.int32),
        ],
    )(x, bsum, W_enc)


def kernel(x, W_enc, W_dec, b_pre, b_post, activated_in):
    B = x.shape[0]
    bsum = (b_post + b_pre)[None, :]
    x_normed, bmax3, fidx3 = _encode_binmax(x, bsum, W_enc)

    bmax = jnp.transpose(bmax3, (1, 0, 2)).reshape(B, L_BINS)
    fidx = jnp.transpose(fidx3, (1, 0, 2)).reshape(B, L_BINS)
    weights, border = jax.lax.top_k(bmax, K)
    indices = jnp.take_along_axis(fidx, border, axis=-1)

    decoded = (weights[..., None] * jnp.take(W_dec, indices, axis=0)).sum(-2)
    y_normed = decoded + b_post
    recon_loss = jnp.mean(jnp.square(x_normed - y_normed), axis=-1)
    y = y_normed / TGT
    loss = recon_loss
    fvu = jnp.mean(jnp.square(x_normed - y_normed)) / jnp.mean(jnp.square(x_normed))
    correlation = ((x_normed - x_normed.mean(axis=0)) * (y_normed - y_normed.mean(axis=0))).mean(axis=0)
    var_explained = (jnp.square(correlation) / (jnp.var(x_normed, axis=0) * jnp.var(y_normed, axis=0))).mean()
    return (y, weights, indices, loss, fvu, var_explained)


# speedup vs baseline: 1.0101x; 1.0101x over previous
import functools
import math

import jax
import jax.numpy as jnp
from jax import lax
from jax.experimental import pallas as pl
from jax.experimental.pallas import tpu as pltpu

D_MODEL = 768
N_FEATURES = 49152
K = 32
CLIP_DATA = 10.0
TGT = math.sqrt(D_MODEL)

BT = 256
FT = 2048
NG = 6
GW = N_FEATURES // NG
NT = GW // FT
L_BINS = NG * 128


def _enc_body(x_ref, bsum_ref, w_ref, xn_ref, bmax_ref, fidx_ref,
              wbf_scr, macc_scr, facc_scr):
    g = pl.program_id(0)
    tt = pl.program_id(1)
    b = pl.program_id(2)

    @pl.when(b == 0)
    def _cast_w():
        wbf_scr[...] = w_ref[...].astype(jnp.bfloat16)

    xn = jnp.clip(x_ref[pl.ds(b * BT, BT), :] * TGT, -CLIP_DATA, CLIP_DATA)
    xn_ref[...] = xn
    xm = (xn - bsum_ref[...]).astype(jnp.bfloat16)
    enc = jnp.dot(xm, wbf_scr[...], preferred_element_type=jnp.float32)
    m = enc[:, 0:128]
    a = jnp.zeros((BT, 128), jnp.int32)
    for s in range(1, 16):
        sl = enc[:, s * 128:(s + 1) * 128]
        upd = sl > m
        m = jnp.where(upd, sl, m)
        a = jnp.where(upd, s, a)
    c = lax.broadcasted_iota(jnp.int32, (BT, 128), 1)
    fidx = g * GW + (tt * 16 + a) * 128 + c

    @pl.when(tt == 0)
    def _init():
        macc_scr[b] = m
        facc_scr[b] = fidx

    @pl.when(tt != 0)
    def _acc():
        prev = macc_scr[b]
        upd2 = m > prev
        macc_scr[b] = jnp.where(upd2, m, prev)
        facc_scr[b] = jnp.where(upd2, fidx, facc_scr[b])

    @pl.when(tt == NT - 1)
    def _emit():
        bmax_ref[...] = macc_scr[b][None]
        fidx_ref[...] = facc_scr[b][None]


def _encode_binmax(x, bsum, W_enc):
    B = x.shape[0]
    NB = B // BT
    grid = (NG, NT, NB)
    return pl.pallas_call(
        _enc_body,
        grid=grid,
        in_specs=[
            pl.BlockSpec((B, D_MODEL), lambda g, t, b: (0, 0)),
            pl.BlockSpec((1, D_MODEL), lambda g, t, b: (0, 0)),
            pl.BlockSpec((D_MODEL, FT), lambda g, t, b: (0, g * NT + t)),
        ],
        out_specs=[
            pl.BlockSpec((BT, D_MODEL), lambda g, t, b: (b, 0)),
            pl.BlockSpec((1, BT, 128), lambda g, t, b: (g, b, 0)),
            pl.BlockSpec((1, BT, 128), lambda g, t, b: (g, b, 0)),
        ],
        out_shape=[
            jax.ShapeDtypeStruct((B, D_MODEL), jnp.float32),
            jax.ShapeDtypeStruct((NG, B, 128), jnp.float32),
            jax.ShapeDtypeStruct((NG, B, 128), jnp.int32),
        ],
        scratch_shapes=[
            pltpu.VMEM((D_MODEL, FT), jnp.bfloat16),
            pltpu.VMEM((B // BT, BT, 128), jnp.float32),
            pltpu.VMEM((B // BT, BT, 128), jnp.int32),
        ],
    )(x, bsum, W_enc)


def kernel(x, W_enc, W_dec, b_pre, b_post, activated_in):
    B = x.shape[0]
    bsum = (b_post + b_pre)[None, :]
    x_normed, bmax3, fidx3 = _encode_binmax(x, bsum, W_enc)

    bmax = jnp.transpose(bmax3, (1, 0, 2)).reshape(B, L_BINS)
    fidx = jnp.transpose(fidx3, (1, 0, 2)).reshape(B, L_BINS)
    weights, border = jax.lax.top_k(bmax, K)
    indices = jnp.take_along_axis(fidx, border, axis=-1)

    decoded = (weights[..., None] * jnp.take(W_dec, indices, axis=0)).sum(-2)
    y_normed = decoded + b_post
    recon_loss = jnp.mean(jnp.square(x_normed - y_normed), axis=-1)
    y = y_normed / TGT
    loss = recon_loss
    fvu = jnp.mean(jnp.square(x_normed - y_normed)) / jnp.mean(jnp.square(x_normed))
    correlation = ((x_normed - x_normed.mean(axis=0)) * (y_normed - y_normed.mean(axis=0))).mean(axis=0)
    var_explained = (jnp.square(correlation) / (jnp.var(x_normed, axis=0) * jnp.var(y_normed, axis=0))).mean()
    return (y, weights, indices, loss, fvu, var_explained)


# speedup vs baseline: 1.2490x; 1.2365x over previous
import functools
import math

import jax
import jax.numpy as jnp
from jax import lax
from jax.experimental import pallas as pl
from jax.experimental.pallas import tpu as pltpu
from jax.experimental.pallas import tpu_sc as plsc

D_MODEL = 768
N_FEATURES = 49152
K = 32
CLIP_DATA = 10.0
TGT = math.sqrt(D_MODEL)

BT = 256
FT = 2048
NG = 6
GW = N_FEATURES // NG
NT = GW // FT
L_BINS = NG * 128

NW = 32
CCH = D_MODEL // 16


def _enc_body(x_ref, bsum_ref, w_ref, xn_ref, bmax_ref, fidx_ref,
              wbf_scr, macc_scr, facc_scr):
    g = pl.program_id(0)
    tt = pl.program_id(1)
    b = pl.program_id(2)

    @pl.when(b == 0)
    def _cast_w():
        wbf_scr[...] = w_ref[...].astype(jnp.bfloat16)

    xn = jnp.clip(x_ref[pl.ds(b * BT, BT), :] * TGT, -CLIP_DATA, CLIP_DATA)
    xn_ref[...] = xn
    xm = (xn - bsum_ref[...]).astype(jnp.bfloat16)
    enc = jnp.dot(xm, wbf_scr[...], preferred_element_type=jnp.float32)
    m = enc[:, 0:128]
    a = jnp.zeros((BT, 128), jnp.int32)
    for s in range(1, 16):
        sl = enc[:, s * 128:(s + 1) * 128]
        upd = sl > m
        m = jnp.where(upd, sl, m)
        a = jnp.where(upd, s, a)
    c = lax.broadcasted_iota(jnp.int32, (BT, 128), 1)
    fidx = g * GW + (tt * 16 + a) * 128 + c

    @pl.when(tt == 0)
    def _init():
        macc_scr[b] = m
        facc_scr[b] = fidx

    @pl.when(tt != 0)
    def _acc():
        prev = macc_scr[b]
        upd2 = m > prev
        macc_scr[b] = jnp.where(upd2, m, prev)
        facc_scr[b] = jnp.where(upd2, fidx, facc_scr[b])

    @pl.when(tt == NT - 1)
    def _emit():
        bmax_ref[...] = macc_scr[b]
        fidx_ref[...] = facc_scr[b]


def _encode_binmax(x, bsum, W_enc):
    B = x.shape[0]
    NB = B // BT
    grid = (NG, NT, NB)
    return pl.pallas_call(
        _enc_body,
        grid=grid,
        in_specs=[
            pl.BlockSpec((B, D_MODEL), lambda g, t, b: (0, 0)),
            pl.BlockSpec((1, D_MODEL), lambda g, t, b: (0, 0)),
            pl.BlockSpec((D_MODEL, FT), lambda g, t, b: (0, g * NT + t)),
        ],
        out_specs=[
            pl.BlockSpec((BT, D_MODEL), lambda g, t, b: (b, 0)),
            pl.BlockSpec((BT, 128), lambda g, t, b: (b, g)),
            pl.BlockSpec((BT, 128), lambda g, t, b: (b, g)),
        ],
        out_shape=[
            jax.ShapeDtypeStruct((B, D_MODEL), jnp.float32),
            jax.ShapeDtypeStruct((B, L_BINS), jnp.float32),
            jax.ShapeDtypeStruct((B, L_BINS), jnp.int32),
        ],
        scratch_shapes=[
            pltpu.VMEM((D_MODEL, FT), jnp.bfloat16),
            pltpu.VMEM((B // BT, BT, 128), jnp.float32),
            pltpu.VMEM((B // BT, BT, 128), jnp.int32),
        ],
    )(x, bsum, W_enc)


def _make_decode(B):
    TPW = B // NW
    mesh = plsc.VectorSubcoreMesh(core_axis_name="c", subcore_axis_name="s")

    @functools.partial(
        pl.kernel, mesh=mesh,
        out_type=jax.ShapeDtypeStruct((B, D_MODEL), jnp.float32),
        scratch_types=[
            pltpu.VMEM((TPW, K), jnp.int32),
            pltpu.VMEM((TPW * K * 16,), jnp.float32),
            pltpu.VMEM((2, K, D_MODEL), jnp.float32),
            pltpu.VMEM((D_MODEL,), jnp.float32),
            pltpu.SemaphoreType.DMA,
            pltpu.SemaphoreType.DMA,
        ],
    )
    def _decode(idx_hbm, w_hbm, wdec_hbm, out_hbm, idx_v, w_v, rows_v, acc_v,
                sem0, sem1):
        wid = lax.axis_index("s") * 2 + lax.axis_index("c")
        base = wid * TPW
        pltpu.sync_copy(idx_hbm.at[pl.ds(base, TPW)], idx_v)
        pltpu.sync_copy(w_hbm.at[pl.ds(base * K * 16, TPW * K * 16)], w_v)
        pltpu.async_copy(wdec_hbm.at[idx_v.at[0]], rows_v.at[0], sem0)
        pltpu.async_copy(wdec_hbm.at[idx_v.at[1]], rows_v.at[1], sem1)

        def compute_token(i, slot, sem):
            pltpu.make_async_copy(wdec_hbm.at[idx_v.at[i]], rows_v.at[slot], sem).wait()

            def kstep(k, acc):
                wk = w_v[pl.ds(i * (K * 16) + k * 16, 16)]
                return tuple(
                    acc[cc] + wk * rows_v[slot, k, pl.ds(cc * 16, 16)]
                    for cc in range(CCH))

            acc0 = tuple(jnp.zeros((16,), jnp.float32) for _ in range(CCH))
            acc = lax.fori_loop(0, K, kstep, acc0)
            for cc in range(CCH):
                acc_v[pl.ds(cc * 16, 16)] = acc[cc]
            pltpu.sync_copy(acc_v, out_hbm.at[base + i])

        def pair(p, carry):
            i0 = 2 * p
            compute_token(i0, 0, sem0)

            @pl.when(i0 + 2 < TPW)
            def _():
                pltpu.async_copy(wdec_hbm.at[idx_v.at[i0 + 2]], rows_v.at[0], sem0)

            compute_token(i0 + 1, 1, sem1)

            @pl.when(i0 + 3 < TPW)
            def _():
                pltpu.async_copy(wdec_hbm.at[idx_v.at[i0 + 3]], rows_v.at[1], sem1)

            return carry

        lax.fori_loop(0, TPW // 2, pair, 0)

    return _decode


def kernel(x, W_enc, W_dec, b_pre, b_post, activated_in):
    B = x.shape[0]
    bsum = (b_post + b_pre)[None, :]
    x_normed, bmax, fidx = _encode_binmax(x, bsum, W_enc)

    weights, border = jax.lax.top_k(bmax, K)
    indices = jnp.take_along_axis(fidx, border, axis=-1)

    w_rep = jnp.broadcast_to(weights[..., None], (B, K, 16)).reshape(-1)
    decoded = _make_decode(B)(indices, w_rep, W_dec)
    y_normed = decoded + b_post
    recon_loss = jnp.mean(jnp.square(x_normed - y_normed), axis=-1)
    y = y_normed / TGT
    loss = recon_loss
    fvu = jnp.mean(jnp.square(x_normed - y_normed)) / jnp.mean(jnp.square(x_normed))
    correlation = ((x_normed - x_normed.mean(axis=0)) * (y_normed - y_normed.mean(axis=0))).mean(axis=0)
    var_explained = (jnp.square(correlation) / (jnp.var(x_normed, axis=0) * jnp.var(y_normed, axis=0))).mean()
    return (y, weights, indices, loss, fvu, var_explained)


# speedup vs baseline: 1.7545x; 1.4047x over previous
import functools
import math

import jax
import jax.numpy as jnp
from jax import lax
from jax.experimental import pallas as pl
from jax.experimental.pallas import tpu as pltpu
from jax.experimental.pallas import tpu_sc as plsc

D_MODEL = 768
N_FEATURES = 49152
K = 32
CLIP_DATA = 10.0
TGT = math.sqrt(D_MODEL)

BT = 256
FT = 2048
NG = 6
GW = N_FEATURES // NG
NT = GW // FT
L_BINS = NG * 128

NW = 32
CCH = D_MODEL // 16


def _enc_body(x_ref, bsum_ref, w_ref, xn_ref, bmax_ref, fidx_ref,
              wbf_scr, macc_scr, facc_scr):
    g = pl.program_id(0)
    tt = pl.program_id(1)
    b = pl.program_id(2)

    @pl.when(b == 0)
    def _cast_w():
        wbf_scr[...] = w_ref[...].astype(jnp.bfloat16)

    xn = jnp.clip(x_ref[pl.ds(b * BT, BT), :] * TGT, -CLIP_DATA, CLIP_DATA)
    xn_ref[...] = xn
    xm = (xn - bsum_ref[...]).astype(jnp.bfloat16)
    enc = jnp.dot(xm, wbf_scr[...], preferred_element_type=jnp.float32)
    m = enc[:, 0:128]
    a = jnp.zeros((BT, 128), jnp.int32)
    for s in range(1, 16):
        sl = enc[:, s * 128:(s + 1) * 128]
        upd = sl > m
        m = jnp.where(upd, sl, m)
        a = jnp.where(upd, s, a)
    c = lax.broadcasted_iota(jnp.int32, (BT, 128), 1)
    fidx = g * GW + (tt * 16 + a) * 128 + c

    @pl.when(tt == 0)
    def _init():
        macc_scr[b] = m
        facc_scr[b] = fidx

    @pl.when(tt != 0)
    def _acc():
        prev = macc_scr[b]
        upd2 = m > prev
        macc_scr[b] = jnp.where(upd2, m, prev)
        facc_scr[b] = jnp.where(upd2, fidx, facc_scr[b])

    @pl.when(tt == NT - 1)
    def _emit():
        bmax_ref[...] = macc_scr[b]
        fidx_ref[...] = facc_scr[b]


def _encode_binmax(x, bsum, W_enc):
    B = x.shape[0]
    NB = B // BT
    grid = (NG, NT, NB)
    return pl.pallas_call(
        _enc_body,
        grid=grid,
        in_specs=[
            pl.BlockSpec((B, D_MODEL), lambda g, t, b: (0, 0)),
            pl.BlockSpec((1, D_MODEL), lambda g, t, b: (0, 0)),
            pl.BlockSpec((D_MODEL, FT), lambda g, t, b: (0, g * NT + t)),
        ],
        out_specs=[
            pl.BlockSpec((BT, D_MODEL), lambda g, t, b: (b, 0)),
            pl.BlockSpec((BT, 128), lambda g, t, b: (b, g)),
            pl.BlockSpec((BT, 128), lambda g, t, b: (b, g)),
        ],
        out_shape=[
            jax.ShapeDtypeStruct((B, D_MODEL), jnp.float32),
            jax.ShapeDtypeStruct((B, L_BINS), jnp.float32),
            jax.ShapeDtypeStruct((B, L_BINS), jnp.int32),
        ],
        scratch_shapes=[
            pltpu.VMEM((D_MODEL, FT), jnp.bfloat16),
            pltpu.VMEM((B // BT, BT, 128), jnp.float32),
            pltpu.VMEM((B // BT, BT, 128), jnp.int32),
        ],
    )(x, bsum, W_enc)


BTB = 256


def _topk_body(bmax_ref, fidx_ref, w_ref, i_ref):
    vals = bmax_ref[...]
    fidxb = fidx_ref[...]
    jiota = lax.broadcasted_iota(jnp.int32, (BTB, L_BINS), 1)
    for k in range(K):
        m = jnp.max(vals, axis=1, keepdims=True)
        eq = vals == m
        cand = jnp.where(eq, jiota, L_BINS)
        jmin = jnp.min(cand, axis=1, keepdims=True)
        sel = jiota == jmin
        w_ref[:, k:k + 1] = m
        i_ref[:, k:k + 1] = jnp.sum(jnp.where(sel, fidxb, 0), axis=1, keepdims=True)
        vals = jnp.where(sel, -jnp.inf, vals)


def _topk(bmax, fidx):
    B = bmax.shape[0]
    return pl.pallas_call(
        _topk_body,
        grid=(B // BTB,),
        in_specs=[
            pl.BlockSpec((BTB, L_BINS), lambda b: (b, 0)),
            pl.BlockSpec((BTB, L_BINS), lambda b: (b, 0)),
        ],
        out_specs=[
            pl.BlockSpec((BTB, K), lambda b: (b, 0)),
            pl.BlockSpec((BTB, K), lambda b: (b, 0)),
        ],
        out_shape=[
            jax.ShapeDtypeStruct((B, K), jnp.float32),
            jax.ShapeDtypeStruct((B, K), jnp.int32),
        ],
    )(bmax, fidx)


def _make_decode(B):
    TPW = B // NW
    mesh = plsc.VectorSubcoreMesh(core_axis_name="c", subcore_axis_name="s")

    @functools.partial(
        pl.kernel, mesh=mesh,
        out_type=jax.ShapeDtypeStruct((B, D_MODEL), jnp.float32),
        scratch_types=[
            pltpu.VMEM((TPW, K), jnp.int32),
            pltpu.VMEM((TPW * K * 16,), jnp.float32),
            pltpu.VMEM((2, K, D_MODEL), jnp.float32),
            pltpu.VMEM((D_MODEL,), jnp.float32),
            pltpu.SemaphoreType.DMA,
            pltpu.SemaphoreType.DMA,
        ],
    )
    def _decode(idx_hbm, w_hbm, wdec_hbm, out_hbm, idx_v, w_v, rows_v, acc_v,
                sem0, sem1):
        wid = lax.axis_index("s") * 2 + lax.axis_index("c")
        base = wid * TPW
        pltpu.sync_copy(idx_hbm.at[pl.ds(base, TPW)], idx_v)
        pltpu.sync_copy(w_hbm.at[pl.ds(base * K * 16, TPW * K * 16)], w_v)
        pltpu.async_copy(wdec_hbm.at[idx_v.at[0]], rows_v.at[0], sem0)
        pltpu.async_copy(wdec_hbm.at[idx_v.at[1]], rows_v.at[1], sem1)

        def compute_token(i, slot, sem):
            pltpu.make_async_copy(wdec_hbm.at[idx_v.at[i]], rows_v.at[slot], sem).wait()

            def kstep(k, acc):
                wk = w_v[pl.ds(i * (K * 16) + k * 16, 16)]
                return tuple(
                    acc[cc] + wk * rows_v[slot, k, pl.ds(cc * 16, 16)]
                    for cc in range(CCH))

            acc0 = tuple(jnp.zeros((16,), jnp.float32) for _ in range(CCH))
            acc = lax.fori_loop(0, K, kstep, acc0)
            for cc in range(CCH):
                acc_v[pl.ds(cc * 16, 16)] = acc[cc]
            pltpu.sync_copy(acc_v, out_hbm.at[base + i])

        def pair(p, carry):
            i0 = 2 * p
            compute_token(i0, 0, sem0)

            @pl.when(i0 + 2 < TPW)
            def _():
                pltpu.async_copy(wdec_hbm.at[idx_v.at[i0 + 2]], rows_v.at[0], sem0)

            compute_token(i0 + 1, 1, sem1)

            @pl.when(i0 + 3 < TPW)
            def _():
                pltpu.async_copy(wdec_hbm.at[idx_v.at[i0 + 3]], rows_v.at[1], sem1)

            return carry

        lax.fori_loop(0, TPW // 2, pair, 0)

    return _decode


def kernel(x, W_enc, W_dec, b_pre, b_post, activated_in):
    B = x.shape[0]
    bsum = (b_post + b_pre)[None, :]
    x_normed, bmax, fidx = _encode_binmax(x, bsum, W_enc)

    weights, indices = _topk(bmax, fidx)

    w_rep = jnp.broadcast_to(weights[..., None], (B, K, 16)).reshape(-1)
    decoded = _make_decode(B)(indices, w_rep, W_dec)
    y_normed = decoded + b_post
    recon_loss = jnp.mean(jnp.square(x_normed - y_normed), axis=-1)
    y = y_normed / TGT
    loss = recon_loss
    fvu = jnp.mean(jnp.square(x_normed - y_normed)) / jnp.mean(jnp.square(x_normed))
    correlation = ((x_normed - x_normed.mean(axis=0)) * (y_normed - y_normed.mean(axis=0))).mean(axis=0)
    var_explained = (jnp.square(correlation) / (jnp.var(x_normed, axis=0) * jnp.var(y_normed, axis=0))).mean()
    return (y, weights, indices, loss, fvu, var_explained)


# speedup vs baseline: 1.7812x; 1.0152x over previous
import functools
import math

import jax
import jax.numpy as jnp
from jax import lax
from jax.experimental import pallas as pl
from jax.experimental.pallas import tpu as pltpu
from jax.experimental.pallas import tpu_sc as plsc

D_MODEL = 768
N_FEATURES = 49152
K = 32
CLIP_DATA = 10.0
TGT = math.sqrt(D_MODEL)

BT = 256
FT = 2048
NG = 6
GW = N_FEATURES // NG
NT = GW // FT
L_BINS = NG * 128

NW = 32
CCH = D_MODEL // 16


def _enc_body(x_ref, bsum_ref, w_ref, xn_ref, bmax_ref, fidx_ref,
              wbf_scr, macc_scr, facc_scr):
    g = pl.program_id(0)
    tt = pl.program_id(1)
    b = pl.program_id(2)

    @pl.when(b == 0)
    def _cast_w():
        wbf_scr[...] = w_ref[...].astype(jnp.bfloat16)

    xn = jnp.clip(x_ref[pl.ds(b * BT, BT), :] * TGT, -CLIP_DATA, CLIP_DATA)
    xn_ref[...] = xn
    xm = (xn - bsum_ref[...]).astype(jnp.bfloat16)
    enc = jnp.dot(xm, wbf_scr[...], preferred_element_type=jnp.float32)
    m = enc[:, 0:128]
    a = jnp.zeros((BT, 128), jnp.int32)
    for s in range(1, 16):
        sl = enc[:, s * 128:(s + 1) * 128]
        upd = sl > m
        m = jnp.where(upd, sl, m)
        a = jnp.where(upd, s, a)
    c = lax.broadcasted_iota(jnp.int32, (BT, 128), 1)
    fidx = g * GW + (tt * 16 + a) * 128 + c

    @pl.when(tt == 0)
    def _init():
        macc_scr[b] = m
        facc_scr[b] = fidx

    @pl.when(tt != 0)
    def _acc():
        prev = macc_scr[b]
        upd2 = m > prev
        macc_scr[b] = jnp.where(upd2, m, prev)
        facc_scr[b] = jnp.where(upd2, fidx, facc_scr[b])

    @pl.when(tt == NT - 1)
    def _emit():
        bmax_ref[...] = macc_scr[b]
        fidx_ref[...] = facc_scr[b]


def _encode_binmax(x, bsum, W_enc):
    B = x.shape[0]
    NB = B // BT
    grid = (NG, NT, NB)
    return pl.pallas_call(
        _enc_body,
        grid=grid,
        in_specs=[
            pl.BlockSpec((B, D_MODEL), lambda g, t, b: (0, 0)),
            pl.BlockSpec((1, D_MODEL), lambda g, t, b: (0, 0)),
            pl.BlockSpec((D_MODEL, FT), lambda g, t, b: (0, g * NT + t)),
        ],
        out_specs=[
            pl.BlockSpec((BT, D_MODEL), lambda g, t, b: (b, 0)),
            pl.BlockSpec((BT, 128), lambda g, t, b: (b, g)),
            pl.BlockSpec((BT, 128), lambda g, t, b: (b, g)),
        ],
        out_shape=[
            jax.ShapeDtypeStruct((B, D_MODEL), jnp.float32),
            jax.ShapeDtypeStruct((B, L_BINS), jnp.float32),
            jax.ShapeDtypeStruct((B, L_BINS), jnp.int32),
        ],
        scratch_shapes=[
            pltpu.VMEM((D_MODEL, FT), jnp.bfloat16),
            pltpu.VMEM((B // BT, BT, 128), jnp.float32),
            pltpu.VMEM((B // BT, BT, 128), jnp.int32),
        ],
    )(x, bsum, W_enc)


BTB = 256


def _topk_body(bmax_ref, fidx_ref, w_ref, i_ref):
    vals = bmax_ref[...]
    fidxb = fidx_ref[...]
    jiota = lax.broadcasted_iota(jnp.int32, (BTB, L_BINS), 1)
    for k in range(K):
        m = jnp.max(vals, axis=1, keepdims=True)
        eq = vals == m
        cand = jnp.where(eq, jiota, L_BINS)
        jmin = jnp.min(cand, axis=1, keepdims=True)
        sel = jiota == jmin
        w_ref[:, k:k + 1] = m
        i_ref[:, k:k + 1] = jnp.sum(jnp.where(sel, fidxb, 0), axis=1, keepdims=True)
        vals = jnp.where(sel, -jnp.inf, vals)


def _topk(bmax, fidx):
    B = bmax.shape[0]
    return pl.pallas_call(
        _topk_body,
        grid=(B // BTB,),
        in_specs=[
            pl.BlockSpec((BTB, L_BINS), lambda b: (b, 0)),
            pl.BlockSpec((BTB, L_BINS), lambda b: (b, 0)),
        ],
        out_specs=[
            pl.BlockSpec((BTB, K), lambda b: (b, 0)),
            pl.BlockSpec((BTB, K), lambda b: (b, 0)),
        ],
        out_shape=[
            jax.ShapeDtypeStruct((B, K), jnp.float32),
            jax.ShapeDtypeStruct((B, K), jnp.int32),
        ],
    )(bmax, fidx)


def _make_decode(B):
    TPW = B // NW
    mesh = plsc.VectorSubcoreMesh(core_axis_name="c", subcore_axis_name="s")

    @functools.partial(
        pl.kernel, mesh=mesh,
        out_type=jax.ShapeDtypeStruct((B, D_MODEL), jnp.float32),
        scratch_types=[
            pltpu.VMEM((TPW, K), jnp.int32),
            pltpu.VMEM((TPW * K * 16,), jnp.float32),
            pltpu.VMEM((2, K, D_MODEL), jnp.float32),
            pltpu.VMEM((D_MODEL,), jnp.float32),
            pltpu.SemaphoreType.DMA,
            pltpu.SemaphoreType.DMA,
        ],
    )
    def _decode(idx_hbm, w_hbm, wdec_hbm, out_hbm, idx_v, w_v, rows_v, acc_v,
                sem0, sem1):
        wid = lax.axis_index("s") * 2 + lax.axis_index("c")
        base = wid * TPW
        pltpu.sync_copy(idx_hbm.at[pl.ds(base, TPW)], idx_v)
        pltpu.sync_copy(w_hbm.at[pl.ds(base * K * 16, TPW * K * 16)], w_v)
        pltpu.async_copy(wdec_hbm.at[idx_v.at[0]], rows_v.at[0], sem0)
        pltpu.async_copy(wdec_hbm.at[idx_v.at[1]], rows_v.at[1], sem1)

        def compute_token(i, slot, sem):
            pltpu.make_async_copy(wdec_hbm.at[idx_v.at[i]], rows_v.at[slot], sem).wait()

            def kstep(k, acc):
                wk = w_v[pl.ds(i * (K * 16) + k * 16, 16)]
                return tuple(
                    acc[cc] + wk * rows_v[slot, k, pl.ds(cc * 16, 16)]
                    for cc in range(CCH))

            acc0 = tuple(jnp.zeros((16,), jnp.float32) for _ in range(CCH))
            acc = lax.fori_loop(0, K, kstep, acc0)
            for cc in range(CCH):
                acc_v[pl.ds(cc * 16, 16)] = acc[cc]
            pltpu.sync_copy(acc_v, out_hbm.at[base + i])

        def pair(p, carry):
            i0 = 2 * p
            compute_token(i0, 0, sem0)

            @pl.when(i0 + 2 < TPW)
            def _():
                pltpu.async_copy(wdec_hbm.at[idx_v.at[i0 + 2]], rows_v.at[0], sem0)

            compute_token(i0 + 1, 1, sem1)

            @pl.when(i0 + 3 < TPW)
            def _():
                pltpu.async_copy(wdec_hbm.at[idx_v.at[i0 + 3]], rows_v.at[1], sem1)

            return carry

        lax.fori_loop(0, TPW // 2, pair, 0)

    return _decode


def _stats_body(xn_ref, dec_ref, bpost_ref, y_ref, loss_ref, fvu_ref, ve_ref, acc_scr):
    b = pl.program_id(0)
    nb = pl.num_programs(0)
    xn = xn_ref[...]
    yn = dec_ref[...] + bpost_ref[...]
    y_ref[...] = yn / TGT
    sq = jnp.square(xn - yn)
    loss_ref[...] = jnp.mean(sq, axis=1)

    sums = jnp.concatenate([
        jnp.sum(xn, axis=0, keepdims=True),
        jnp.sum(yn, axis=0, keepdims=True),
        jnp.sum(xn * yn, axis=0, keepdims=True),
        jnp.sum(jnp.square(xn), axis=0, keepdims=True),
        jnp.sum(jnp.square(yn), axis=0, keepdims=True),
        jnp.sum(sq, axis=0, keepdims=True),
    ], axis=0)

    @pl.when(b == 0)
    def _():
        acc_scr[...] = sums

    @pl.when(b != 0)
    def _():
        acc_scr[...] = acc_scr[...] + sums

    @pl.when(b == nb - 1)
    def _():
        tot = acc_scr[...]
        B = nb * BT * 1.0
        sx = tot[0:1]
        sy = tot[1:2]
        sxy = tot[2:3]
        sx2 = tot[3:4]
        sy2 = tot[4:5]
        serr = tot[5:6]
        corr = sxy / B - (sx / B) * (sy / B)
        varx = sx2 / B - jnp.square(sx / B)
        vary = sy2 / B - jnp.square(sy / B)
        ve = jnp.square(corr) / (varx * vary)
        fvu_ref[...] = (jnp.sum(serr, axis=1, keepdims=True)
                        / jnp.sum(sx2, axis=1, keepdims=True))
        ve_ref[...] = jnp.sum(ve, axis=1, keepdims=True) / D_MODEL


def _stats(xn, dec, b_post):
    B = xn.shape[0]
    NB = B // BT
    return pl.pallas_call(
        _stats_body,
        grid=(NB,),
        in_specs=[
            pl.BlockSpec((BT, D_MODEL), lambda b: (b, 0)),
            pl.BlockSpec((BT, D_MODEL), lambda b: (b, 0)),
            pl.BlockSpec((1, D_MODEL), lambda b: (0, 0)),
        ],
        out_specs=[
            pl.BlockSpec((BT, D_MODEL), lambda b: (b, 0)),
            pl.BlockSpec((BT,), lambda b: (b,)),
            pl.BlockSpec((1, 1), lambda b: (0, 0)),
            pl.BlockSpec((1, 1), lambda b: (0, 0)),
        ],
        out_shape=[
            jax.ShapeDtypeStruct((B, D_MODEL), jnp.float32),
            jax.ShapeDtypeStruct((B,), jnp.float32),
            jax.ShapeDtypeStruct((1, 1), jnp.float32),
            jax.ShapeDtypeStruct((1, 1), jnp.float32),
        ],
        scratch_shapes=[
            pltpu.VMEM((6, D_MODEL), jnp.float32),
        ],
    )(xn, dec, b_post)


def kernel(x, W_enc, W_dec, b_pre, b_post, activated_in):
    B = x.shape[0]
    bsum = (b_post + b_pre)[None, :]
    x_normed, bmax, fidx = _encode_binmax(x, bsum, W_enc)

    weights, indices = _topk(bmax, fidx)

    w_rep = jnp.broadcast_to(weights[..., None], (B, K, 16)).reshape(-1)
    decoded = _make_decode(B)(indices, w_rep, W_dec)
    y, loss, fvu, var_explained = _stats(x_normed, decoded, b_post[None, :])
    return (y, weights, indices, loss, fvu.reshape(()), var_explained.reshape(()))


# speedup vs baseline: 1.9196x; 1.0777x over previous
import functools
import math

import jax
import jax.numpy as jnp
from jax import lax
from jax.experimental import pallas as pl
from jax.experimental.pallas import tpu as pltpu
from jax.experimental.pallas import tpu_sc as plsc

D_MODEL = 768
N_FEATURES = 49152
K = 32
CLIP_DATA = 10.0
TGT = math.sqrt(D_MODEL)

BT = 256
BTA = 512
FT = 2048
NG = 6
GW = N_FEATURES // NG
NT = GW // FT
L_BINS = NG * 128

NW = 32
CCH = D_MODEL // 16


def _enc_body(x_ref, bsum_ref, w_ref, xn_ref, bmax_ref, fidx_ref,
              wbf_scr, macc_scr, facc_scr):
    g = pl.program_id(0)
    tt = pl.program_id(1)
    b = pl.program_id(2)

    @pl.when(b == 0)
    def _cast_w():
        wbf_scr[...] = w_ref[...].astype(jnp.bfloat16)

    xn = jnp.clip(x_ref[pl.ds(b * BTA, BTA), :] * TGT, -CLIP_DATA, CLIP_DATA)
    xn_ref[...] = xn
    xm = (xn - bsum_ref[...]).astype(jnp.bfloat16)
    enc = jnp.dot(xm, wbf_scr[...], preferred_element_type=jnp.float32)
    m = enc[:, 0:128]
    a = jnp.zeros((BTA, 128), jnp.int32)
    for s in range(1, 16):
        sl = enc[:, s * 128:(s + 1) * 128]
        upd = sl > m
        m = jnp.where(upd, sl, m)
        a = jnp.where(upd, s, a)
    c = lax.broadcasted_iota(jnp.int32, (BTA, 128), 1)
    fidx = g * GW + (tt * 16 + a) * 128 + c

    @pl.when(tt == 0)
    def _init():
        macc_scr[b] = m
        facc_scr[b] = fidx

    @pl.when(tt != 0)
    def _acc():
        prev = macc_scr[b]
        upd2 = m > prev
        macc_scr[b] = jnp.where(upd2, m, prev)
        facc_scr[b] = jnp.where(upd2, fidx, facc_scr[b])

    @pl.when(tt == NT - 1)
    def _emit():
        bmax_ref[...] = macc_scr[b]
        fidx_ref[...] = facc_scr[b]


def _encode_binmax(x, bsum, W_enc):
    B = x.shape[0]
    NB = B // BTA
    grid = (NG, NT, NB)
    return pl.pallas_call(
        _enc_body,
        grid=grid,
        in_specs=[
            pl.BlockSpec((B, D_MODEL), lambda g, t, b: (0, 0)),
            pl.BlockSpec((1, D_MODEL), lambda g, t, b: (0, 0)),
            pl.BlockSpec((D_MODEL, FT), lambda g, t, b: (0, g * NT + t)),
        ],
        out_specs=[
            pl.BlockSpec((BTA, D_MODEL), lambda g, t, b: (b, 0)),
            pl.BlockSpec((BTA, 128), lambda g, t, b: (b, g)),
            pl.BlockSpec((BTA, 128), lambda g, t, b: (b, g)),
        ],
        out_shape=[
            jax.ShapeDtypeStruct((B, D_MODEL), jnp.float32),
            jax.ShapeDtypeStruct((B, L_BINS), jnp.float32),
            jax.ShapeDtypeStruct((B, L_BINS), jnp.int32),
        ],
        scratch_shapes=[
            pltpu.VMEM((D_MODEL, FT), jnp.bfloat16),
            pltpu.VMEM((B // BTA, BTA, 128), jnp.float32),
            pltpu.VMEM((B // BTA, BTA, 128), jnp.int32),
        ],
    )(x, bsum, W_enc)


BTB = 256


def _topk_body(bmax_ref, fidx_ref, w_ref, i_ref):
    vals = bmax_ref[...]
    fidxb = fidx_ref[...]
    jiota = lax.broadcasted_iota(jnp.int32, (BTB, L_BINS), 1)
    for k in range(K):
        m = jnp.max(vals, axis=1, keepdims=True)
        eq = vals == m
        cand = jnp.where(eq, jiota, L_BINS)
        jmin = jnp.min(cand, axis=1, keepdims=True)
        sel = jiota == jmin
        w_ref[:, k:k + 1] = m
        i_ref[:, k:k + 1] = jnp.sum(jnp.where(sel, fidxb, 0), axis=1, keepdims=True)
        vals = jnp.where(sel, -jnp.inf, vals)


def _topk(bmax, fidx):
    B = bmax.shape[0]
    return pl.pallas_call(
        _topk_body,
        grid=(B // BTB,),
        in_specs=[
            pl.BlockSpec((BTB, L_BINS), lambda b: (b, 0)),
            pl.BlockSpec((BTB, L_BINS), lambda b: (b, 0)),
        ],
        out_specs=[
            pl.BlockSpec((BTB, K), lambda b: (b, 0)),
            pl.BlockSpec((BTB, K), lambda b: (b, 0)),
        ],
        out_shape=[
            jax.ShapeDtypeStruct((B, K), jnp.float32),
            jax.ShapeDtypeStruct((B, K), jnp.int32),
        ],
    )(bmax, fidx)


def _make_decode(B):
    TPW = B // NW
    mesh = plsc.VectorSubcoreMesh(core_axis_name="c", subcore_axis_name="s")

    @functools.partial(
        pl.kernel, mesh=mesh,
        out_type=jax.ShapeDtypeStruct((B, D_MODEL), jnp.float32),
        scratch_types=[
            pltpu.VMEM((TPW, K), jnp.int32),
            pltpu.VMEM((TPW * K * 16,), jnp.float32),
            pltpu.VMEM((2, K, D_MODEL), jnp.float32),
            pltpu.VMEM((D_MODEL,), jnp.float32),
            pltpu.SemaphoreType.DMA,
            pltpu.SemaphoreType.DMA,
        ],
    )
    def _decode(idx_hbm, w_hbm, wdec_hbm, out_hbm, idx_v, w_v, rows_v, acc_v,
                sem0, sem1):
        wid = lax.axis_index("s") * 2 + lax.axis_index("c")
        base = wid * TPW
        pltpu.sync_copy(idx_hbm.at[pl.ds(base, TPW)], idx_v)
        pltpu.sync_copy(w_hbm.at[pl.ds(base * K * 16, TPW * K * 16)], w_v)
        pltpu.async_copy(wdec_hbm.at[idx_v.at[0]], rows_v.at[0], sem0)
        pltpu.async_copy(wdec_hbm.at[idx_v.at[1]], rows_v.at[1], sem1)

        def compute_token(i, slot, sem):
            pltpu.make_async_copy(wdec_hbm.at[idx_v.at[i]], rows_v.at[slot], sem).wait()

            def kstep(k, acc):
                wk = w_v[pl.ds(i * (K * 16) + k * 16, 16)]
                return tuple(
                    acc[cc] + wk * rows_v[slot, k, pl.ds(cc * 16, 16)]
                    for cc in range(CCH))

            acc0 = tuple(jnp.zeros((16,), jnp.float32) for _ in range(CCH))
            acc = lax.fori_loop(0, K, kstep, acc0)
            for cc in range(CCH):
                acc_v[pl.ds(cc * 16, 16)] = acc[cc]
            pltpu.sync_copy(acc_v, out_hbm.at[base + i])

        def pair(p, carry):
            i0 = 2 * p
            compute_token(i0, 0, sem0)

            @pl.when(i0 + 2 < TPW)
            def _():
                pltpu.async_copy(wdec_hbm.at[idx_v.at[i0 + 2]], rows_v.at[0], sem0)

            compute_token(i0 + 1, 1, sem1)

            @pl.when(i0 + 3 < TPW)
            def _():
                pltpu.async_copy(wdec_hbm.at[idx_v.at[i0 + 3]], rows_v.at[1], sem1)

            return carry

        lax.fori_loop(0, TPW // 2, pair, 0)

    return _decode


def _stats_body(xn_ref, dec_ref, bpost_ref, y_ref, loss_ref, fvu_ref, ve_ref, acc_scr):
    b = pl.program_id(0)
    nb = pl.num_programs(0)
    xn = xn_ref[...]
    yn = dec_ref[...] + bpost_ref[...]
    y_ref[...] = yn / TGT
    sq = jnp.square(xn - yn)
    loss_ref[...] = jnp.mean(sq, axis=1)

    sums = jnp.concatenate([
        jnp.sum(xn, axis=0, keepdims=True),
        jnp.sum(yn, axis=0, keepdims=True),
        jnp.sum(xn * yn, axis=0, keepdims=True),
        jnp.sum(jnp.square(xn), axis=0, keepdims=True),
        jnp.sum(jnp.square(yn), axis=0, keepdims=True),
        jnp.sum(sq, axis=0, keepdims=True),
    ], axis=0)

    @pl.when(b == 0)
    def _():
        acc_scr[...] = sums

    @pl.when(b != 0)
    def _():
        acc_scr[...] = acc_scr[...] + sums

    @pl.when(b == nb - 1)
    def _():
        tot = acc_scr[...]
        B = nb * BT * 1.0
        sx = tot[0:1]
        sy = tot[1:2]
        sxy = tot[2:3]
        sx2 = tot[3:4]
        sy2 = tot[4:5]
        serr = tot[5:6]
        corr = sxy / B - (sx / B) * (sy / B)
        varx = sx2 / B - jnp.square(sx / B)
        vary = sy2 / B - jnp.square(sy / B)
        ve = jnp.square(corr) / (varx * vary)
        fvu_ref[...] = (jnp.sum(serr, axis=1, keepdims=True)
                        / jnp.sum(sx2, axis=1, keepdims=True))
        ve_ref[...] = jnp.sum(ve, axis=1, keepdims=True) / D_MODEL


def _stats(xn, dec, b_post):
    B = xn.shape[0]
    NB = B // BT
    return pl.pallas_call(
        _stats_body,
        grid=(NB,),
        in_specs=[
            pl.BlockSpec((BT, D_MODEL), lambda b: (b, 0)),
            pl.BlockSpec((BT, D_MODEL), lambda b: (b, 0)),
            pl.BlockSpec((1, D_MODEL), lambda b: (0, 0)),
        ],
        out_specs=[
            pl.BlockSpec((BT, D_MODEL), lambda b: (b, 0)),
            pl.BlockSpec((BT,), lambda b: (b,)),
            pl.BlockSpec((1, 1), lambda b: (0, 0)),
            pl.BlockSpec((1, 1), lambda b: (0, 0)),
        ],
        out_shape=[
            jax.ShapeDtypeStruct((B, D_MODEL), jnp.float32),
            jax.ShapeDtypeStruct((B,), jnp.float32),
            jax.ShapeDtypeStruct((1, 1), jnp.float32),
            jax.ShapeDtypeStruct((1, 1), jnp.float32),
        ],
        scratch_shapes=[
            pltpu.VMEM((6, D_MODEL), jnp.float32),
        ],
    )(xn, dec, b_post)


def kernel(x, W_enc, W_dec, b_pre, b_post, activated_in):
    B = x.shape[0]
    bsum = (b_post + b_pre)[None, :]
    x_normed, bmax, fidx = _encode_binmax(x, bsum, W_enc)

    weights, indices = _topk(bmax, fidx)

    w_rep = jnp.broadcast_to(weights[..., None], (B, K, 16)).reshape(-1)
    decoded = _make_decode(B)(indices, w_rep, W_dec)
    y, loss, fvu, var_explained = _stats(x_normed, decoded, b_post[None, :])
    return (y, weights, indices, loss, fvu.reshape(()), var_explained.reshape(()))
